# baseline (speedup 1.0000x reference)
import functools
import math

import jax
import jax.numpy as jnp
import numpy as np
from jax import lax
from jax.experimental import pallas as pl
from jax.experimental.pallas import tpu as pltpu
from jax.experimental.pallas import tpu_sc as plsc

H_B, W_B = 512, 512
H_R, W_R = 64, 2048
P = H_B * W_B
B = 2
C = 64
Z_MIN, Z_MAX = -4.0, 2.0
Z_BINS = 30
PHI_MIN, PHI_MAX = -math.pi, math.pi
THETA_MIN, THETA_MAX = math.radians(-25.0), math.radians(3.0)
XMIN, XMAX, YMIN, YMAX = -50.0, 50.0, -50.0, 50.0

NUM_WORKERS = 32
UNIT_COLS = 16
NUM_CU = W_R // UNIT_COLS
CHUNK_PAIRS = 64
CHUNK_PX = CHUNK_PAIRS * 2


def _static_prep():
    y = np.linspace(YMAX, YMIN, H_B)
    x = np.linspace(XMIN, XMAX, W_B)
    yg, xg = np.meshgrid(y, x, indexing="ij")
    phi = np.arctan2(yg, xg)
    col = np.clip(np.round((phi - PHI_MIN) / (PHI_MAX - PHI_MIN) * (W_R - 1)),
                  0, W_R - 1).astype(np.int32).ravel()
    rho = (np.sqrt(xg ** 2 + yg ** 2) + 1e-6).astype(np.float32).ravel()

    order = np.argsort(col, kind="stable")
    sorted_cols = col[order]
    starts = np.searchsorted(sorted_cols, np.arange(W_R))
    ends = np.searchsorted(sorted_cols, np.arange(W_R), side="right")

    plist_parts = []
    coff = np.zeros(NUM_CU, np.int64)
    ccnt = np.zeros(NUM_CU, np.int64)
    pair_col = col.reshape(-1, 2)
    total = 0
    for cu in range(NUM_CU):
        c0 = cu * UNIT_COLS
        px = np.concatenate(
            [order[starts[c]:ends[c]] for c in range(c0, c0 + UNIT_COLS)])
        prs = np.unique(px // 2)
        pc = (c0 + W_R // 2) % W_R
        in_unit = lambda cc: c0 <= cc < c0 + UNIT_COLS
        pad_pr = -1
        for cand in range(P // 2):
            cand_pr = (pc * 64 + cand) % (P // 2)
            cc = pair_col[cand_pr]
            if not in_unit(cc[0]) and not in_unit(cc[1]):
                pad_pr = cand_pr
                break
        n = len(prs)
        nck = max(1, -(-n // CHUNK_PAIRS))
        padded = np.full(nck * CHUNK_PAIRS, pad_pr, np.int64)
        padded[:n] = prs
        plist_parts.append(padded)
        coff[cu] = total
        ccnt[cu] = nck
        total += nck
    plist = np.concatenate(plist_parts).astype(np.int32)

    units = [(int(ccnt[cu]), b, cu) for b in range(B) for cu in range(NUM_CU)]
    units.sort(reverse=True)
    loads = [0] * NUM_WORKERS
    slots = [[] for _ in range(NUM_WORKERS)]
    for g, b, cu in units:
        w = int(np.argmin(loads))
        loads[w] += g
        slots[w].append((b, cu))
    nslots = max(len(s) for s in slots)
    sched = np.zeros((nslots * NUM_WORKERS, 16), np.int32)
    for w in range(NUM_WORKERS):
        for s in range(nslots):
            r = s * NUM_WORKERS + w
            if s < len(slots[w]):
                b, cu = slots[w][s]
                sched[r, 0] = coff[cu]
                sched[r, 1] = ccnt[cu]
                sched[r, 2] = cu
                sched[r, 3] = b
            else:
                sched[r, 2] = -1
    return col, rho, plist, sched, nslots


_COL_NP, _RHO_NP, _PLIST_NP, _SCHED_NP, NSLOTS = _static_prep()


def _idx_body(z_ref, rho_ref, col_ref, o_ref):
    dz = (Z_MAX - Z_MIN) / Z_BINS
    z = z_ref[...].astype(jnp.float32) * dz + (Z_MIN + dz / 2.0)
    theta = jnp.arctan2(z, rho_ref[...])
    sc = (H_R - 1) / (THETA_MAX - THETA_MIN)
    row = jnp.clip(jnp.round((THETA_MAX - theta) * sc), 0, H_R - 1).astype(jnp.int32)
    o_ref[...] = row * W_R + col_ref[...]


def _compute_idx(zflat, rho, colv):
    rows, cols = 16, P // 16
    return pl.pallas_call(
        _idx_body,
        grid=(B,),
        in_specs=[
            pl.BlockSpec((rows, cols), lambda b: (b, 0)),
            pl.BlockSpec((rows, cols), lambda b: (0, 0)),
            pl.BlockSpec((rows, cols), lambda b: (0, 0)),
        ],
        out_specs=pl.BlockSpec((rows, cols), lambda b: (b, 0)),
        out_shape=jax.ShapeDtypeStruct((B * rows, cols), jnp.int32),
    )(zflat.reshape(B * rows, cols), rho.reshape(rows, cols),
      colv.reshape(rows, cols)).reshape(B, P)


def _tr2_body(f_ref, o_ref):
    blk = f_ref.shape[1]
    o_ref[...] = jnp.swapaxes(f_ref[...], 0, 1).reshape(blk // 2, 2 * C)


def _sc_scatter_max(featT, idxs, plist, sched):
    mesh = plsc.VectorSubcoreMesh(core_axis_name="c", subcore_axis_name="s")
    lanes = 16
    lpair = plist.shape[0]

    nacc = C * H_R * UNIT_COLS // 128

    @functools.partial(
        pl.kernel,
        mesh=mesh,
        compiler_params=pltpu.CompilerParams(needs_layout_passes=False),
        out_type=jax.ShapeDtypeStruct((B * NUM_CU * nacc * 128,), jnp.float32),
        scratch_types=[
            pltpu.VMEM(((nacc + 1) * 128,), jnp.float32),
            pltpu.VMEM((CHUNK_PAIRS,), jnp.int32),
            pltpu.VMEM((CHUNK_PAIRS,), jnp.int32),
            pltpu.VMEM((CHUNK_PX,), jnp.int32),
            pltpu.VMEM((CHUNK_PAIRS, 2 * C), jnp.float32),
            pltpu.VMEM((lanes,), jnp.int32),
            pltpu.SemaphoreType.DMA,
        ],
    )
    def k(featT_hbm, idx_hbm, plist_hbm, sched_hbm, rv_hbm,
          acc, plbuf, pairbuf, idxbuf, featbuf, schedbuf, sem1):
        wid = lax.axis_index("s") * 2 + lax.axis_index("c")
        ninf = jnp.full((lanes,), -jnp.inf, jnp.float32)
        dummy0 = nacc * 128

        chv = [(jnp.arange(lanes, dtype=jnp.int32) + kb * lanes)
               * (H_R * UNIT_COLS) for kb in range(C // lanes)]

        def slot_body(s, _):
            pltpu.sync_copy(sched_hbm.at[s * NUM_WORKERS + wid], schedbuf)
            sv = schedbuf[...]
            coff = sv[0]
            ccnt = sv[1]
            cu = sv[2]
            bb = sv[3]
            c0 = cu * UNIT_COLS

            @pl.when(cu >= 0)
            def _():
                def init_body(i, _):
                    for u in range(128 // lanes):
                        acc[pl.ds(i * 128 + u * lanes, lanes)] = ninf
                    return 0

                lax.fori_loop(0, nacc + 1, init_body, 0)

                def chunk_body(ci, _):
                    pltpu.sync_copy(
                        plist_hbm.at[
                            pl.ds((coff + ci) * CHUNK_PAIRS, CHUNK_PAIRS)],
                        plbuf)
                    for j in range(CHUNK_PAIRS // lanes):
                        pairbuf[pl.ds(j * lanes, lanes)] = (
                            plbuf[pl.ds(j * lanes, lanes)] + bb * (P // 2))
                    cp1 = pltpu.async_copy(featT_hbm.at[pairbuf], featbuf,
                                           sem1)
                    pltpu.sync_copy(
                        idx_hbm.at[pl.ds(bb * lpair * 2
                                         + (coff + ci) * CHUNK_PX, CHUNK_PX)],
                        idxbuf)
                    cp1.wait()

                    def group_body(j, _):
                        iv = idxbuf[pl.ds(j * lanes, lanes)]
                        for l in range(lanes):
                            binv = iv[l]
                            rowp = lax.shift_right_logical(binv, 11)
                            collp = (binv & (W_R - 1)) - c0
                            valid = (collp >= 0) & (collp < UNIT_COLS)
                            base = jnp.where(valid,
                                             rowp * UNIT_COLS + collp, dummy0)
                            vm = jnp.where(valid, 1, 0)
                            pair = (j * lanes + l) // 2
                            half = l % 2
                            for kb in range(C // lanes):
                                fv = featbuf[pair,
                                             pl.ds(half * C + kb * lanes,
                                                   lanes)]
                                o = chv[kb] * vm + base
                                cur = plsc.load_gather(acc, [o])
                                plsc.store_scatter(acc, [o],
                                                   jnp.maximum(cur, fv))
                        return 0

                    lax.fori_loop(0, CHUNK_PX // lanes, group_body, 0)
                    return 0

                lax.fori_loop(0, ccnt, chunk_body, 0)
                nw = nacc * 128
                pltpu.sync_copy(acc.at[pl.ds(0, nw)],
                                rv_hbm.at[pl.ds((bb * NUM_CU + cu) * nw, nw)])
            return 0

        lax.fori_loop(0, NSLOTS, slot_body, 0)

    return k(featT, idxs, plist, sched)


def _clean_body(r_ref, o_ref):
    v = r_ref[...]
    o_ref[...] = jnp.where(jnp.isneginf(v), jnp.zeros_like(v), v)


def _cleanup(rv):
    blk = 2048
    n = rv.size
    shape = rv.shape
    return pl.pallas_call(
        _clean_body,
        grid=(n // (8 * blk),),
        in_specs=[pl.BlockSpec((8, blk), lambda i: (i, 0))],
        out_specs=pl.BlockSpec((8, blk), lambda i: (i, 0)),
        out_shape=jax.ShapeDtypeStruct((n // blk, blk), jnp.float32),
    )(rv.reshape(n // blk, blk)).reshape(shape)


def _geometry():
    y_lin = jnp.linspace(YMAX, YMIN, H_B)
    x_lin = jnp.linspace(XMIN, XMAX, W_B)
    yg, xg = jnp.meshgrid(y_lin, x_lin, indexing="ij")
    phi = jnp.arctan2(yg, xg)
    colv = jnp.clip(jnp.round((phi - PHI_MIN) / (PHI_MAX - PHI_MIN) * (W_R - 1)),
                    0, W_R - 1).astype(jnp.int32).reshape(-1)
    rho = (jnp.sqrt(xg ** 2 + yg ** 2) + 1e-06).reshape(-1)
    return colv, rho


def kernel(bev_feat, bev_z_bin):
    colv, rho = _geometry()
    zflat = bev_z_bin.reshape(B, P)
    idx = _compute_idx(zflat, rho, colv)

    featT = jnp.swapaxes(bev_feat.reshape(B, C, P), 1, 2).reshape(
        B * P // 2, 2 * C)

    plist = jnp.asarray(_PLIST_NP)
    idxs = idx.reshape(B, P // 2, 2)[:, plist, :].reshape(-1)
    rvt = _sc_scatter_max(featT, idxs, plist, jnp.asarray(_SCHED_NP))
    rvt = _cleanup(rvt)
    return (rvt.reshape(B, NUM_CU, C, H_R, UNIT_COLS)
            .transpose(0, 2, 3, 1, 4).reshape(B, C, H_R, W_R))


# speedup vs baseline: 1.0277x; 1.0277x over previous
import functools
import math

import jax
import jax.numpy as jnp
import numpy as np
from jax import lax
from jax.experimental import pallas as pl
from jax.experimental.pallas import tpu as pltpu
from jax.experimental.pallas import tpu_sc as plsc

H_B, W_B = 512, 512
H_R, W_R = 64, 2048
P = H_B * W_B
B = 2
C = 64
Z_MIN, Z_MAX = -4.0, 2.0
Z_BINS = 30
PHI_MIN, PHI_MAX = -math.pi, math.pi
THETA_MIN, THETA_MAX = math.radians(-25.0), math.radians(3.0)
XMIN, XMAX, YMIN, YMAX = -50.0, 50.0, -50.0, 50.0

NUM_WORKERS = 32
UNIT_COLS = 16
NUM_CU = W_R // UNIT_COLS
CHUNK_PAIRS = 128
CHUNK_PX = CHUNK_PAIRS * 2


def _static_prep():
    y = np.linspace(YMAX, YMIN, H_B)
    x = np.linspace(XMIN, XMAX, W_B)
    yg, xg = np.meshgrid(y, x, indexing="ij")
    phi = np.arctan2(yg, xg)
    col = np.clip(np.round((phi - PHI_MIN) / (PHI_MAX - PHI_MIN) * (W_R - 1)),
                  0, W_R - 1).astype(np.int32).ravel()
    rho = (np.sqrt(xg ** 2 + yg ** 2) + 1e-6).astype(np.float32).ravel()

    order = np.argsort(col, kind="stable")
    sorted_cols = col[order]
    starts = np.searchsorted(sorted_cols, np.arange(W_R))
    ends = np.searchsorted(sorted_cols, np.arange(W_R), side="right")

    plist_parts = []
    coff = np.zeros(NUM_CU, np.int64)
    ccnt = np.zeros(NUM_CU, np.int64)
    pair_col = col.reshape(-1, 2)
    total = 0
    for cu in range(NUM_CU):
        c0 = cu * UNIT_COLS
        px = np.concatenate(
            [order[starts[c]:ends[c]] for c in range(c0, c0 + UNIT_COLS)])
        prs = np.unique(px // 2)
        pc = (c0 + W_R // 2) % W_R
        in_unit = lambda cc: c0 <= cc < c0 + UNIT_COLS
        pad_pr = -1
        for cand in range(P // 2):
            cand_pr = (pc * 64 + cand) % (P // 2)
            cc = pair_col[cand_pr]
            if not in_unit(cc[0]) and not in_unit(cc[1]):
                pad_pr = cand_pr
                break
        n = len(prs)
        nck = max(1, -(-n // CHUNK_PAIRS))
        padded = np.full(nck * CHUNK_PAIRS, pad_pr, np.int64)
        padded[:n] = prs
        plist_parts.append(padded)
        coff[cu] = total
        ccnt[cu] = nck
        total += nck
    plist = np.concatenate(plist_parts).astype(np.int32)

    units = [(int(ccnt[cu]), b, cu) for b in range(B) for cu in range(NUM_CU)]
    units.sort(reverse=True)
    loads = [0] * NUM_WORKERS
    slots = [[] for _ in range(NUM_WORKERS)]
    for g, b, cu in units:
        w = int(np.argmin(loads))
        loads[w] += g
        slots[w].append((b, cu))
    nslots = max(len(s) for s in slots)
    sched = np.zeros((nslots * NUM_WORKERS, 16), np.int32)
    for w in range(NUM_WORKERS):
        for s in range(nslots):
            r = s * NUM_WORKERS + w
            if s < len(slots[w]):
                b, cu = slots[w][s]
                sched[r, 0] = coff[cu]
                sched[r, 1] = ccnt[cu]
                sched[r, 2] = cu
                sched[r, 3] = b
            else:
                sched[r, 2] = -1
    return col, rho, plist, sched, nslots


_COL_NP, _RHO_NP, _PLIST_NP, _SCHED_NP, NSLOTS = _static_prep()


def _idx_body(z_ref, rho_ref, col_ref, o_ref):
    dz = (Z_MAX - Z_MIN) / Z_BINS
    z = z_ref[...].astype(jnp.float32) * dz + (Z_MIN + dz / 2.0)
    theta = jnp.arctan2(z, rho_ref[...])
    sc = (H_R - 1) / (THETA_MAX - THETA_MIN)
    row = jnp.clip(jnp.round((THETA_MAX - theta) * sc), 0, H_R - 1).astype(jnp.int32)
    o_ref[...] = row * W_R + col_ref[...]


def _compute_idx(zflat, rho, colv):
    rows, cols = 16, P // 16
    return pl.pallas_call(
        _idx_body,
        grid=(B,),
        in_specs=[
            pl.BlockSpec((rows, cols), lambda b: (b, 0)),
            pl.BlockSpec((rows, cols), lambda b: (0, 0)),
            pl.BlockSpec((rows, cols), lambda b: (0, 0)),
        ],
        out_specs=pl.BlockSpec((rows, cols), lambda b: (b, 0)),
        out_shape=jax.ShapeDtypeStruct((B * rows, cols), jnp.int32),
    )(zflat.reshape(B * rows, cols), rho.reshape(rows, cols),
      colv.reshape(rows, cols)).reshape(B, P)


def _tr2_body(f_ref, o_ref):
    blk = f_ref.shape[1]
    o_ref[...] = jnp.swapaxes(f_ref[...], 0, 1).reshape(blk // 2, 2 * C)


def _sc_scatter_max(featT, idxs, plist, sched):
    mesh = plsc.VectorSubcoreMesh(core_axis_name="c", subcore_axis_name="s")
    lanes = 16
    lpair = plist.shape[0]

    nacc = C * H_R * UNIT_COLS // 128
    nkb = C // lanes
    qwords = H_R * UNIT_COLS * lanes

    @functools.partial(
        pl.kernel,
        mesh=mesh,
        compiler_params=pltpu.CompilerParams(needs_layout_passes=False),
        out_type=jax.ShapeDtypeStruct((B * NUM_CU * nacc * 128,), jnp.float32),
        scratch_types=[
            [pltpu.VMEM((qwords + 128,), jnp.float32) for _ in range(nkb)],
            pltpu.VMEM((CHUNK_PAIRS,), jnp.int32),
            pltpu.VMEM((CHUNK_PAIRS,), jnp.int32),
            pltpu.VMEM((CHUNK_PX,), jnp.int32),
            pltpu.VMEM((CHUNK_PAIRS, 2 * C), jnp.float32),
            pltpu.VMEM((lanes,), jnp.int32),
            pltpu.SemaphoreType.DMA,
        ],
    )
    def k(featT_hbm, idx_hbm, plist_hbm, sched_hbm, rv_hbm,
          accs, plbuf, pairbuf, idxbuf, featbuf, schedbuf, sem1):
        wid = lax.axis_index("s") * 2 + lax.axis_index("c")
        ninf = jnp.full((lanes,), -jnp.inf, jnp.float32)
        dummy0 = qwords

        chv = jnp.arange(lanes, dtype=jnp.int32) * (H_R * UNIT_COLS)

        def slot_body(s, _):
            pltpu.sync_copy(sched_hbm.at[s * NUM_WORKERS + wid], schedbuf)
            sv = schedbuf[...]
            coff = sv[0]
            ccnt = sv[1]
            cu = sv[2]
            bb = sv[3]
            c0 = cu * UNIT_COLS

            @pl.when(cu >= 0)
            def _():
                def init_body(i, _):
                    for a in accs:
                        for u in range(128 // lanes):
                            a[pl.ds(i * 128 + u * lanes, lanes)] = ninf
                    return 0

                lax.fori_loop(0, qwords // 128 + 1, init_body, 0)

                def chunk_body(ci, _):
                    pltpu.sync_copy(
                        plist_hbm.at[
                            pl.ds((coff + ci) * CHUNK_PAIRS, CHUNK_PAIRS)],
                        plbuf)
                    for j in range(CHUNK_PAIRS // lanes):
                        pairbuf[pl.ds(j * lanes, lanes)] = (
                            plbuf[pl.ds(j * lanes, lanes)] + bb * (P // 2))
                    cp1 = pltpu.async_copy(featT_hbm.at[pairbuf], featbuf,
                                           sem1)
                    pltpu.sync_copy(
                        idx_hbm.at[pl.ds(bb * lpair * 2
                                         + (coff + ci) * CHUNK_PX, CHUNK_PX)],
                        idxbuf)
                    cp1.wait()

                    def group_body(j, _):
                        iv = idxbuf[pl.ds(j * lanes, lanes)]
                        for l in range(lanes):
                            binv = iv[l]
                            rowp = lax.shift_right_logical(binv, 11)
                            collp = (binv & (W_R - 1)) - c0
                            valid = (collp >= 0) & (collp < UNIT_COLS)
                            base = jnp.where(valid,
                                             rowp * UNIT_COLS + collp, dummy0)
                            vm = jnp.where(valid, 1, 0)
                            o = chv * vm + base
                            pair = (j * lanes + l) // 2
                            half = l % 2
                            fvs = [featbuf[pair,
                                           pl.ds(half * C + kb * lanes, lanes)]
                                   for kb in range(nkb)]
                            curs = [plsc.load_gather(accs[kb], [o])
                                    for kb in range(nkb)]
                            for kb in range(nkb):
                                plsc.store_scatter(
                                    accs[kb], [o],
                                    jnp.maximum(curs[kb], fvs[kb]))
                        return 0

                    lax.fori_loop(0, CHUNK_PX // lanes, group_body, 0)
                    return 0

                lax.fori_loop(0, ccnt, chunk_body, 0)
                ubase = (bb * NUM_CU + cu) * (nacc * 128)
                for kb in range(nkb):
                    pltpu.sync_copy(
                        accs[kb].at[pl.ds(0, qwords)],
                        rv_hbm.at[pl.ds(ubase + kb * qwords, qwords)])
            return 0

        lax.fori_loop(0, NSLOTS, slot_body, 0)

    return k(featT, idxs, plist, sched)


def _clean_body(r_ref, o_ref):
    v = r_ref[...]
    o_ref[...] = jnp.where(jnp.isneginf(v), jnp.zeros_like(v), v)


def _cleanup(rv):
    blk = 2048
    n = rv.size
    shape = rv.shape
    return pl.pallas_call(
        _clean_body,
        grid=(n // (8 * blk),),
        in_specs=[pl.BlockSpec((8, blk), lambda i: (i, 0))],
        out_specs=pl.BlockSpec((8, blk), lambda i: (i, 0)),
        out_shape=jax.ShapeDtypeStruct((n // blk, blk), jnp.float32),
    )(rv.reshape(n // blk, blk)).reshape(shape)


def _geometry():
    y_lin = jnp.linspace(YMAX, YMIN, H_B)
    x_lin = jnp.linspace(XMIN, XMAX, W_B)
    yg, xg = jnp.meshgrid(y_lin, x_lin, indexing="ij")
    phi = jnp.arctan2(yg, xg)
    colv = jnp.clip(jnp.round((phi - PHI_MIN) / (PHI_MAX - PHI_MIN) * (W_R - 1)),
                    0, W_R - 1).astype(jnp.int32).reshape(-1)
    rho = (jnp.sqrt(xg ** 2 + yg ** 2) + 1e-06).reshape(-1)
    return colv, rho


def kernel(bev_feat, bev_z_bin):
    colv, rho = _geometry()
    zflat = bev_z_bin.reshape(B, P)
    idx = _compute_idx(zflat, rho, colv)

    featT = jnp.swapaxes(bev_feat.reshape(B, C, P), 1, 2).reshape(
        B * P // 2, 2 * C)

    plist = jnp.asarray(_PLIST_NP)
    idxs = idx.reshape(B, P // 2, 2)[:, plist, :].reshape(-1)
    rvt = _sc_scatter_max(featT, idxs, plist, jnp.asarray(_SCHED_NP))
    rvt = _cleanup(rvt)
    return (rvt.reshape(B, NUM_CU, C, H_R, UNIT_COLS)
            .transpose(0, 2, 3, 1, 4).reshape(B, C, H_R, W_R))


# speedup vs baseline: 1.0347x; 1.0068x over previous
import functools
import math

import jax
import jax.numpy as jnp
import numpy as np
from jax import lax
from jax.experimental import pallas as pl
from jax.experimental.pallas import tpu as pltpu
from jax.experimental.pallas import tpu_sc as plsc

H_B, W_B = 512, 512
H_R, W_R = 64, 2048
P = H_B * W_B
B = 2
C = 64
Z_MIN, Z_MAX = -4.0, 2.0
Z_BINS = 30
PHI_MIN, PHI_MAX = -math.pi, math.pi
THETA_MIN, THETA_MAX = math.radians(-25.0), math.radians(3.0)
XMIN, XMAX, YMIN, YMAX = -50.0, 50.0, -50.0, 50.0

NUM_WORKERS = 32
UNIT_COLS = 16
NUM_CU = W_R // UNIT_COLS
CHUNK_PAIRS = 128
CHUNK_PX = CHUNK_PAIRS * 2


def _static_prep():
    y = np.linspace(YMAX, YMIN, H_B)
    x = np.linspace(XMIN, XMAX, W_B)
    yg, xg = np.meshgrid(y, x, indexing="ij")
    phi = np.arctan2(yg, xg)
    col = np.clip(np.round((phi - PHI_MIN) / (PHI_MAX - PHI_MIN) * (W_R - 1)),
                  0, W_R - 1).astype(np.int32).ravel()
    rho = (np.sqrt(xg ** 2 + yg ** 2) + 1e-6).astype(np.float32).ravel()

    order = np.argsort(col, kind="stable")
    sorted_cols = col[order]
    starts = np.searchsorted(sorted_cols, np.arange(W_R))
    ends = np.searchsorted(sorted_cols, np.arange(W_R), side="right")

    plist_parts = []
    coff = np.zeros(NUM_CU, np.int64)
    ccnt = np.zeros(NUM_CU, np.int64)
    pair_col = col.reshape(-1, 2)
    total = 0
    for cu in range(NUM_CU):
        c0 = cu * UNIT_COLS
        px = np.concatenate(
            [order[starts[c]:ends[c]] for c in range(c0, c0 + UNIT_COLS)])
        prs = np.unique(px // 2)
        pc = (c0 + W_R // 2) % W_R
        in_unit = lambda cc: c0 <= cc < c0 + UNIT_COLS
        pad_pr = -1
        for cand in range(P // 2):
            cand_pr = (pc * 64 + cand) % (P // 2)
            cc = pair_col[cand_pr]
            if not in_unit(cc[0]) and not in_unit(cc[1]):
                pad_pr = cand_pr
                break
        n = len(prs)
        nck = max(1, -(-n // CHUNK_PAIRS))
        padded = np.full(nck * CHUNK_PAIRS, pad_pr, np.int64)
        padded[:n] = prs
        plist_parts.append(padded)
        coff[cu] = total
        ccnt[cu] = nck
        total += nck
    maxcc = int(ccnt.max())
    plist_parts.append(np.full(maxcc * CHUNK_PAIRS, plist_parts[-1][-1],
                               np.int64))
    plist = np.concatenate(plist_parts).astype(np.int32)

    units = [(int(ccnt[cu]), b, cu) for b in range(B) for cu in range(NUM_CU)]
    units.sort(reverse=True)
    loads = [0] * NUM_WORKERS
    slots = [[] for _ in range(NUM_WORKERS)]
    for g, b, cu in units:
        w = int(np.argmin(loads))
        loads[w] += g
        slots[w].append((b, cu))
    nslots = max(len(s) for s in slots)
    sched = np.zeros((nslots * NUM_WORKERS, 16), np.int32)
    for w in range(NUM_WORKERS):
        for s in range(nslots):
            r = s * NUM_WORKERS + w
            if s < len(slots[w]):
                b, cu = slots[w][s]
                sched[r, 0] = coff[cu]
                sched[r, 1] = ccnt[cu]
                sched[r, 2] = cu
                sched[r, 3] = b
            else:
                sched[r, 2] = -1
    return col, rho, plist, sched, nslots, maxcc


_COL_NP, _RHO_NP, _PLIST_NP, _SCHED_NP, NSLOTS, MAXCC = _static_prep()


def _idx_body(z_ref, rho_ref, col_ref, o_ref):
    dz = (Z_MAX - Z_MIN) / Z_BINS
    z = z_ref[...].astype(jnp.float32) * dz + (Z_MIN + dz / 2.0)
    theta = jnp.arctan2(z, rho_ref[...])
    sc = (H_R - 1) / (THETA_MAX - THETA_MIN)
    row = jnp.clip(jnp.round((THETA_MAX - theta) * sc), 0, H_R - 1).astype(jnp.int32)
    o_ref[...] = row * W_R + col_ref[...]


def _compute_idx(zflat, rho, colv):
    rows, cols = 16, P // 16
    return pl.pallas_call(
        _idx_body,
        grid=(B,),
        in_specs=[
            pl.BlockSpec((rows, cols), lambda b: (b, 0)),
            pl.BlockSpec((rows, cols), lambda b: (0, 0)),
            pl.BlockSpec((rows, cols), lambda b: (0, 0)),
        ],
        out_specs=pl.BlockSpec((rows, cols), lambda b: (b, 0)),
        out_shape=jax.ShapeDtypeStruct((B * rows, cols), jnp.int32),
    )(zflat.reshape(B * rows, cols), rho.reshape(rows, cols),
      colv.reshape(rows, cols)).reshape(B, P)


def _tr2_body(f_ref, o_ref):
    blk = f_ref.shape[1]
    o_ref[...] = jnp.swapaxes(f_ref[...], 0, 1).reshape(blk // 2, 2 * C)


def _sc_scatter_max(featT, idxs, plist, sched):
    mesh = plsc.VectorSubcoreMesh(core_axis_name="c", subcore_axis_name="s")
    lanes = 16
    lpair = plist.shape[0]

    nacc = C * H_R * UNIT_COLS // 128
    nkb = C // lanes
    qwords = H_R * UNIT_COLS * lanes

    @functools.partial(
        pl.kernel,
        mesh=mesh,
        compiler_params=pltpu.CompilerParams(needs_layout_passes=False),
        out_type=jax.ShapeDtypeStruct((B * NUM_CU * nacc * 128,), jnp.float32),
        scratch_types=[
            [pltpu.VMEM((qwords + 128,), jnp.float32) for _ in range(nkb)],
            pltpu.VMEM((MAXCC * CHUNK_PAIRS,), jnp.int32),
            pltpu.VMEM((MAXCC * CHUNK_PAIRS,), jnp.int32),
            pltpu.VMEM((MAXCC * CHUNK_PX,), jnp.int32),
            [pltpu.VMEM((CHUNK_PAIRS, 2 * C), jnp.float32)
             for _ in range(2)],
            pltpu.VMEM((lanes,), jnp.int32),
            [pltpu.SemaphoreType.DMA for _ in range(2)],
        ],
    )
    def k(featT_hbm, idx_hbm, plist_hbm, sched_hbm, rv_hbm,
          accs, plbuf, pairbuf, idxbuf, featbufs, schedbuf, sems):
        wid = lax.axis_index("s") * 2 + lax.axis_index("c")
        ninf = jnp.full((lanes,), -jnp.inf, jnp.float32)
        dummy0 = qwords

        chv = jnp.arange(lanes, dtype=jnp.int32) * (H_R * UNIT_COLS)

        def slot_body(s, _):
            pltpu.sync_copy(sched_hbm.at[s * NUM_WORKERS + wid], schedbuf)
            sv = schedbuf[...]
            coff = sv[0]
            ccnt = sv[1]
            cu = sv[2]
            bb = sv[3]
            c0 = cu * UNIT_COLS

            @pl.when(cu >= 0)
            def _():
                def init_body(i, _):
                    for a in accs:
                        for u in range(128 // lanes):
                            a[pl.ds(i * 128 + u * lanes, lanes)] = ninf
                    return 0

                lax.fori_loop(0, qwords // 128 + 1, init_body, 0)

                pltpu.sync_copy(
                    plist_hbm.at[pl.ds(coff * CHUNK_PAIRS,
                                       MAXCC * CHUNK_PAIRS)], plbuf)
                pltpu.sync_copy(
                    idx_hbm.at[pl.ds(bb * lpair * 2 + coff * CHUNK_PX,
                                     MAXCC * CHUNK_PX)], idxbuf)

                def off_body(j, _):
                    pairbuf[pl.ds(j * lanes, lanes)] = (
                        plbuf[pl.ds(j * lanes, lanes)] + bb * (P // 2))
                    return 0

                lax.fori_loop(0, MAXCC * CHUNK_PAIRS // lanes, off_body, 0)

                def issue(ci, buf, sem):
                    return pltpu.async_copy(
                        featT_hbm.at[
                            pairbuf.at[pl.ds(ci * CHUNK_PAIRS, CHUNK_PAIRS)]],
                        buf, sem)

                issue(0, featbufs[0], sems[0])

                def compute(ci, featbuf):
                    def group_body(j, _):
                        iv = idxbuf[pl.ds(ci * CHUNK_PX + j * lanes, lanes)]
                        rowv = lax.shift_right_logical(iv, 11)
                        collv = (iv & (W_R - 1)) - c0
                        validm = (collv >= 0) & (collv < UNIT_COLS)
                        basev = jnp.where(validm, rowv * UNIT_COLS + collv,
                                          dummy0)
                        vmv = jnp.where(validm, jnp.ones_like(iv),
                                        jnp.zeros_like(iv))
                        for l in range(lanes):
                            o = chv * vmv[l] + basev[l]
                            pair = j * (lanes // 2) + l // 2
                            fvs = [featbuf[pair,
                                           pl.ds((l % 2) * C + kb * lanes,
                                                 lanes)]
                                   for kb in range(nkb)]
                            curs = [plsc.load_gather(accs[kb], [o])
                                    for kb in range(nkb)]
                            for kb in range(nkb):
                                plsc.store_scatter(
                                    accs[kb], [o],
                                    jnp.maximum(curs[kb], fvs[kb]))
                        return 0

                    lax.fori_loop(0, CHUNK_PX // lanes, group_body, 0)

                def chunk_pair(ci2, _):
                    ci0 = ci2 * 2

                    @pl.when(ci0 + 1 < ccnt)
                    def _():
                        issue(ci0 + 1, featbufs[1], sems[1])

                    pltpu.make_async_copy(
                        featT_hbm.at[pairbuf.at[pl.ds(0, CHUNK_PAIRS)]],
                        featbufs[0], sems[0]).wait()
                    compute(ci0, featbufs[0])

                    @pl.when(ci0 + 1 < ccnt)
                    def _():
                        @pl.when(ci0 + 2 < ccnt)
                        def _():
                            issue(ci0 + 2, featbufs[0], sems[0])

                        pltpu.make_async_copy(
                            featT_hbm.at[pairbuf.at[pl.ds(0, CHUNK_PAIRS)]],
                            featbufs[1], sems[1]).wait()
                        compute(ci0 + 1, featbufs[1])
                    return 0

                lax.fori_loop(0, (ccnt + 1) // 2, chunk_pair, 0)
                ubase = (bb * NUM_CU + cu) * (nacc * 128)
                for kb in range(nkb):
                    pltpu.sync_copy(
                        accs[kb].at[pl.ds(0, qwords)],
                        rv_hbm.at[pl.ds(ubase + kb * qwords, qwords)])
            return 0

        lax.fori_loop(0, NSLOTS, slot_body, 0)

    return k(featT, idxs, plist, sched)


def _clean_body(r_ref, o_ref):
    v = r_ref[...]
    o_ref[...] = jnp.where(jnp.isneginf(v), jnp.zeros_like(v), v)


def _cleanup(rv):
    blk = 2048
    n = rv.size
    shape = rv.shape
    return pl.pallas_call(
        _clean_body,
        grid=(n // (8 * blk),),
        in_specs=[pl.BlockSpec((8, blk), lambda i: (i, 0))],
        out_specs=pl.BlockSpec((8, blk), lambda i: (i, 0)),
        out_shape=jax.ShapeDtypeStruct((n // blk, blk), jnp.float32),
    )(rv.reshape(n // blk, blk)).reshape(shape)


def _geometry():
    y_lin = jnp.linspace(YMAX, YMIN, H_B)
    x_lin = jnp.linspace(XMIN, XMAX, W_B)
    yg, xg = jnp.meshgrid(y_lin, x_lin, indexing="ij")
    phi = jnp.arctan2(yg, xg)
    colv = jnp.clip(jnp.round((phi - PHI_MIN) / (PHI_MAX - PHI_MIN) * (W_R - 1)),
                    0, W_R - 1).astype(jnp.int32).reshape(-1)
    rho = (jnp.sqrt(xg ** 2 + yg ** 2) + 1e-06).reshape(-1)
    return colv, rho


def kernel(bev_feat, bev_z_bin):
    colv, rho = _geometry()
    zflat = bev_z_bin.reshape(B, P)
    idx = _compute_idx(zflat, rho, colv)

    featT = jnp.swapaxes(bev_feat.reshape(B, C, P), 1, 2).reshape(
        B * P // 2, 2 * C)

    plist = jnp.asarray(_PLIST_NP)
    idxs = idx.reshape(B, P // 2, 2)[:, plist, :].reshape(-1)
    rvt = _sc_scatter_max(featT, idxs, plist, jnp.asarray(_SCHED_NP))
    rvt = _cleanup(rvt)
    return (rvt.reshape(B, NUM_CU, C, H_R, UNIT_COLS)
            .transpose(0, 2, 3, 1, 4).reshape(B, C, H_R, W_R))


# speedup vs baseline: 1.4365x; 1.3883x over previous
import functools
import math

import jax
import jax.numpy as jnp
import numpy as np
from jax import lax
from jax.experimental import pallas as pl
from jax.experimental.pallas import tpu as pltpu
from jax.experimental.pallas import tpu_sc as plsc

H_B, W_B = 512, 512
H_R, W_R = 64, 2048
P = H_B * W_B
B = 2
C = 64
Z_MIN, Z_MAX = -4.0, 2.0
Z_BINS = 30
PHI_MIN, PHI_MAX = -math.pi, math.pi
THETA_MIN, THETA_MAX = math.radians(-25.0), math.radians(3.0)
XMIN, XMAX, YMIN, YMAX = -50.0, 50.0, -50.0, 50.0

NUM_WORKERS = 32
UNIT_COLS = 16
NUM_CU = W_R // UNIT_COLS
CHUNK_PAIRS = 128
CHUNK_PX = CHUNK_PAIRS * 2


def _static_prep():
    y = np.linspace(YMAX, YMIN, H_B)
    x = np.linspace(XMIN, XMAX, W_B)
    yg, xg = np.meshgrid(y, x, indexing="ij")
    phi = np.arctan2(yg, xg)
    col = np.clip(np.round((phi - PHI_MIN) / (PHI_MAX - PHI_MIN) * (W_R - 1)),
                  0, W_R - 1).astype(np.int32).ravel()
    rho = (np.sqrt(xg ** 2 + yg ** 2) + 1e-6).astype(np.float32).ravel()

    order = np.argsort(col, kind="stable")
    sorted_cols = col[order]
    starts = np.searchsorted(sorted_cols, np.arange(W_R))
    ends = np.searchsorted(sorted_cols, np.arange(W_R), side="right")

    plist_parts = []
    coff = np.zeros(NUM_CU, np.int64)
    ccnt = np.zeros(NUM_CU, np.int64)
    pair_col = col.reshape(-1, 2)
    total = 0
    for cu in range(NUM_CU):
        c0 = cu * UNIT_COLS
        px = np.concatenate(
            [order[starts[c]:ends[c]] for c in range(c0, c0 + UNIT_COLS)])
        prs = np.unique(px // 2)
        pc = (c0 + W_R // 2) % W_R
        in_unit = lambda cc: c0 <= cc < c0 + UNIT_COLS
        pad_pr = -1
        for cand in range(P // 2):
            cand_pr = (pc * 64 + cand) % (P // 2)
            cc = pair_col[cand_pr]
            if not in_unit(cc[0]) and not in_unit(cc[1]):
                pad_pr = cand_pr
                break
        n = len(prs)
        nck = max(1, -(-n // CHUNK_PAIRS))
        padded = np.full(nck * CHUNK_PAIRS, pad_pr, np.int64)
        padded[:n] = prs
        plist_parts.append(padded)
        coff[cu] = total
        ccnt[cu] = nck
        total += nck
    maxcc = int(ccnt.max())
    plist_parts.append(np.full(maxcc * CHUNK_PAIRS, plist_parts[-1][-1],
                               np.int64))
    plist = np.concatenate(plist_parts).astype(np.int32)

    units = [(int(ccnt[cu]), b, cu) for b in range(B) for cu in range(NUM_CU)]
    units.sort(reverse=True)
    loads = [0] * NUM_WORKERS
    slots = [[] for _ in range(NUM_WORKERS)]
    for g, b, cu in units:
        w = int(np.argmin(loads))
        loads[w] += g
        slots[w].append((b, cu))
    nslots = max(len(s) for s in slots)
    sched = np.zeros((nslots * NUM_WORKERS, 16), np.int32)
    for w in range(NUM_WORKERS):
        for s in range(nslots):
            r = s * NUM_WORKERS + w
            if s < len(slots[w]):
                b, cu = slots[w][s]
                sched[r, 0] = coff[cu]
                sched[r, 1] = ccnt[cu]
                sched[r, 2] = cu
                sched[r, 3] = b
            else:
                sched[r, 2] = -1
    return col, rho, plist, sched, nslots, maxcc


_COL_NP, _RHO_NP, _PLIST_NP, _SCHED_NP, NSLOTS, MAXCC = _static_prep()


def _idx_body(z_ref, rho_ref, col_ref, o_ref):
    dz = (Z_MAX - Z_MIN) / Z_BINS
    z = z_ref[...].astype(jnp.float32) * dz + (Z_MIN + dz / 2.0)
    theta = jnp.arctan2(z, rho_ref[...])
    sc = (H_R - 1) / (THETA_MAX - THETA_MIN)
    row = jnp.clip(jnp.round((THETA_MAX - theta) * sc), 0, H_R - 1).astype(jnp.int32)
    o_ref[...] = row * W_R + col_ref[...]


def _compute_idx(zflat, rho, colv):
    rows, cols = 16, P // 16
    return pl.pallas_call(
        _idx_body,
        grid=(B,),
        in_specs=[
            pl.BlockSpec((rows, cols), lambda b: (b, 0)),
            pl.BlockSpec((rows, cols), lambda b: (0, 0)),
            pl.BlockSpec((rows, cols), lambda b: (0, 0)),
        ],
        out_specs=pl.BlockSpec((rows, cols), lambda b: (b, 0)),
        out_shape=jax.ShapeDtypeStruct((B * rows, cols), jnp.int32),
    )(zflat.reshape(B * rows, cols), rho.reshape(rows, cols),
      colv.reshape(rows, cols)).reshape(B, P)


def _tr2_body(f_ref, o_ref):
    blk = f_ref.shape[1]
    o_ref[...] = jnp.swapaxes(f_ref[...], 0, 1).reshape(blk // 2, 2 * C)


def _sc_scatter_max(featT, idxs, plist, sched):
    mesh = plsc.VectorSubcoreMesh(core_axis_name="c", subcore_axis_name="s")
    lanes = 16
    lpair = plist.shape[0]

    nacc = C * H_R * UNIT_COLS // 128
    nkb = C // lanes
    qwords = H_R * UNIT_COLS * lanes

    @functools.partial(
        pl.kernel,
        mesh=mesh,
        compiler_params=pltpu.CompilerParams(needs_layout_passes=False),
        out_type=jax.ShapeDtypeStruct((B * NUM_CU * nacc * 128,), jnp.float32),
        scratch_types=[
            [pltpu.VMEM((qwords + 128,), jnp.float32) for _ in range(nkb)],
            pltpu.VMEM((MAXCC * CHUNK_PAIRS,), jnp.int32),
            pltpu.VMEM((MAXCC * CHUNK_PAIRS,), jnp.int32),
            pltpu.VMEM((MAXCC * CHUNK_PX,), jnp.int32),
            [pltpu.VMEM((CHUNK_PAIRS, 2 * C), jnp.float32)
             for _ in range(2)],
            pltpu.VMEM((lanes,), jnp.int32),
            [pltpu.SemaphoreType.DMA for _ in range(2)],
        ],
    )
    def k(featT_hbm, idx_hbm, plist_hbm, sched_hbm, rv_hbm,
          accs, plbuf, pairbuf, idxbuf, featbufs, schedbuf, sems):
        wid = lax.axis_index("s") * 2 + lax.axis_index("c")
        ninf = jnp.full((lanes,), -jnp.inf, jnp.float32)
        dummy0 = H_R * UNIT_COLS

        chv = jnp.arange(lanes, dtype=jnp.int32)

        def slot_body(s, _):
            pltpu.sync_copy(sched_hbm.at[s * NUM_WORKERS + wid], schedbuf)
            sv = schedbuf[...]
            coff = sv[0]
            ccnt = sv[1]
            cu = sv[2]
            bb = sv[3]
            c0 = cu * UNIT_COLS

            @pl.when(cu >= 0)
            def _():
                def init_body(i, _):
                    for a in accs:
                        for u in range(128 // lanes):
                            a[pl.ds(i * 128 + u * lanes, lanes)] = ninf
                    return 0

                lax.fori_loop(0, qwords // 128 + 1, init_body, 0)

                pltpu.sync_copy(
                    plist_hbm.at[pl.ds(coff * CHUNK_PAIRS,
                                       MAXCC * CHUNK_PAIRS)], plbuf)
                pltpu.sync_copy(
                    idx_hbm.at[pl.ds(bb * lpair * 2 + coff * CHUNK_PX,
                                     MAXCC * CHUNK_PX)], idxbuf)

                def off_body(j, _):
                    pairbuf[pl.ds(j * lanes, lanes)] = (
                        plbuf[pl.ds(j * lanes, lanes)] + bb * (P // 2))
                    return 0

                lax.fori_loop(0, MAXCC * CHUNK_PAIRS // lanes, off_body, 0)

                def issue(ci, buf, sem):
                    return pltpu.async_copy(
                        featT_hbm.at[
                            pairbuf.at[pl.ds(ci * CHUNK_PAIRS, CHUNK_PAIRS)]],
                        buf, sem)

                issue(0, featbufs[0], sems[0])

                def compute(ci, featbuf):
                    def group_body(j, _):
                        iv = idxbuf[pl.ds(ci * CHUNK_PX + j * lanes, lanes)]
                        rowv = lax.shift_right_logical(iv, 11)
                        collv = (iv & (W_R - 1)) - c0
                        validm = (collv >= 0) & (collv < UNIT_COLS)
                        basev = (jnp.where(validm,
                                           rowv * UNIT_COLS + collv,
                                           dummy0) * lanes)
                        for l in range(lanes):
                            o = chv + basev[l]
                            pair = j * (lanes // 2) + l // 2
                            fvs = [featbuf[pair,
                                           pl.ds((l % 2) * C + kb * lanes,
                                                 lanes)]
                                   for kb in range(nkb)]
                            curs = [plsc.load_gather(accs[kb], [o])
                                    for kb in range(nkb)]
                            for kb in range(nkb):
                                plsc.store_scatter(
                                    accs[kb], [o],
                                    jnp.maximum(curs[kb], fvs[kb]))
                        return 0

                    lax.fori_loop(0, CHUNK_PX // lanes, group_body, 0)

                def chunk_pair(ci2, _):
                    ci0 = ci2 * 2

                    @pl.when(ci0 + 1 < ccnt)
                    def _():
                        issue(ci0 + 1, featbufs[1], sems[1])

                    pltpu.make_async_copy(
                        featT_hbm.at[pairbuf.at[pl.ds(0, CHUNK_PAIRS)]],
                        featbufs[0], sems[0]).wait()
                    compute(ci0, featbufs[0])

                    @pl.when(ci0 + 1 < ccnt)
                    def _():
                        @pl.when(ci0 + 2 < ccnt)
                        def _():
                            issue(ci0 + 2, featbufs[0], sems[0])

                        pltpu.make_async_copy(
                            featT_hbm.at[pairbuf.at[pl.ds(0, CHUNK_PAIRS)]],
                            featbufs[1], sems[1]).wait()
                        compute(ci0 + 1, featbufs[1])
                    return 0

                lax.fori_loop(0, (ccnt + 1) // 2, chunk_pair, 0)
                ubase = (bb * NUM_CU + cu) * (nacc * 128)
                for kb in range(nkb):
                    pltpu.sync_copy(
                        accs[kb].at[pl.ds(0, qwords)],
                        rv_hbm.at[pl.ds(ubase + kb * qwords, qwords)])
            return 0

        lax.fori_loop(0, NSLOTS, slot_body, 0)

    return k(featT, idxs, plist, sched)


def _clean_body(r_ref, o_ref):
    v = r_ref[...]
    o_ref[...] = jnp.where(jnp.isneginf(v), jnp.zeros_like(v), v)


def _cleanup(rv):
    blk = 2048
    n = rv.size
    shape = rv.shape
    return pl.pallas_call(
        _clean_body,
        grid=(n // (8 * blk),),
        in_specs=[pl.BlockSpec((8, blk), lambda i: (i, 0))],
        out_specs=pl.BlockSpec((8, blk), lambda i: (i, 0)),
        out_shape=jax.ShapeDtypeStruct((n // blk, blk), jnp.float32),
    )(rv.reshape(n // blk, blk)).reshape(shape)


def _geometry():
    y_lin = jnp.linspace(YMAX, YMIN, H_B)
    x_lin = jnp.linspace(XMIN, XMAX, W_B)
    yg, xg = jnp.meshgrid(y_lin, x_lin, indexing="ij")
    phi = jnp.arctan2(yg, xg)
    colv = jnp.clip(jnp.round((phi - PHI_MIN) / (PHI_MAX - PHI_MIN) * (W_R - 1)),
                    0, W_R - 1).astype(jnp.int32).reshape(-1)
    rho = (jnp.sqrt(xg ** 2 + yg ** 2) + 1e-06).reshape(-1)
    return colv, rho


def kernel(bev_feat, bev_z_bin):
    colv, rho = _geometry()
    zflat = bev_z_bin.reshape(B, P)
    idx = _compute_idx(zflat, rho, colv)

    featT = jnp.swapaxes(bev_feat.reshape(B, C, P), 1, 2).reshape(
        B * P // 2, 2 * C)

    plist = jnp.asarray(_PLIST_NP)
    idxs = idx.reshape(B, P // 2, 2)[:, plist, :].reshape(-1)
    rvt = _sc_scatter_max(featT, idxs, plist, jnp.asarray(_SCHED_NP))
    rvt = _cleanup(rvt)
    return (rvt.reshape(B, NUM_CU, C // 16, H_R, UNIT_COLS, 16)
            .transpose(0, 2, 5, 3, 1, 4).reshape(B, C, H_R, W_R))


# speedup vs baseline: 2.2165x; 1.5431x over previous
import functools
import math

import jax
import jax.numpy as jnp
import numpy as np
from jax import lax
from jax.experimental import pallas as pl
from jax.experimental.pallas import tpu as pltpu
from jax.experimental.pallas import tpu_sc as plsc

H_B, W_B = 512, 512
H_R, W_R = 64, 2048
P = H_B * W_B
B = 2
C = 64
Z_MIN, Z_MAX = -4.0, 2.0
Z_BINS = 30
PHI_MIN, PHI_MAX = -math.pi, math.pi
THETA_MIN, THETA_MAX = math.radians(-25.0), math.radians(3.0)
XMIN, XMAX, YMIN, YMAX = -50.0, 50.0, -50.0, 50.0

NUM_WORKERS = 32
UNIT_COLS = 16
NUM_CU = W_R // UNIT_COLS
CHUNK_PAIRS = 128
CHUNK_PX = CHUNK_PAIRS * 2


def _static_prep():
    y = np.linspace(YMAX, YMIN, H_B)
    x = np.linspace(XMIN, XMAX, W_B)
    yg, xg = np.meshgrid(y, x, indexing="ij")
    phi = np.arctan2(yg, xg)
    col = np.clip(np.round((phi - PHI_MIN) / (PHI_MAX - PHI_MIN) * (W_R - 1)),
                  0, W_R - 1).astype(np.int32).ravel()
    rho = (np.sqrt(xg ** 2 + yg ** 2) + 1e-6).astype(np.float32).ravel()

    order = np.argsort(col, kind="stable")
    sorted_cols = col[order]
    starts = np.searchsorted(sorted_cols, np.arange(W_R))
    ends = np.searchsorted(sorted_cols, np.arange(W_R), side="right")

    plist_parts = []
    coff = np.zeros(NUM_CU, np.int64)
    ccnt = np.zeros(NUM_CU, np.int64)
    pair_col = col.reshape(-1, 2)
    total = 0
    for cu in range(NUM_CU):
        c0 = cu * UNIT_COLS
        px = np.concatenate(
            [order[starts[c]:ends[c]] for c in range(c0, c0 + UNIT_COLS)])
        prs = np.unique(px // 2)
        pc = (c0 + W_R // 2) % W_R
        in_unit = lambda cc: c0 <= cc < c0 + UNIT_COLS
        pad_pr = -1
        for cand in range(P // 2):
            cand_pr = (pc * 64 + cand) % (P // 2)
            cc = pair_col[cand_pr]
            if not in_unit(cc[0]) and not in_unit(cc[1]):
                pad_pr = cand_pr
                break
        n = len(prs)
        nck = max(1, -(-n // CHUNK_PAIRS))
        padded = np.full(nck * CHUNK_PAIRS, pad_pr, np.int64)
        padded[:n] = prs
        plist_parts.append(padded)
        coff[cu] = total
        ccnt[cu] = nck
        total += nck
    maxcc = int(ccnt.max())
    plist_parts.append(np.full(maxcc * CHUNK_PAIRS, plist_parts[-1][-1],
                               np.int64))
    plist = np.concatenate(plist_parts).astype(np.int32)

    units = [(int(ccnt[cu]), b, cu) for b in range(B) for cu in range(NUM_CU)]
    units.sort(reverse=True)
    loads = [0] * NUM_WORKERS
    slots = [[] for _ in range(NUM_WORKERS)]
    for g, b, cu in units:
        w = int(np.argmin(loads))
        loads[w] += g
        slots[w].append((b, cu))
    nslots = max(len(s) for s in slots)
    sched = np.zeros((nslots * NUM_WORKERS, 16), np.int32)
    for w in range(NUM_WORKERS):
        for s in range(nslots):
            r = s * NUM_WORKERS + w
            if s < len(slots[w]):
                b, cu = slots[w][s]
                sched[r, 0] = coff[cu]
                sched[r, 1] = ccnt[cu]
                sched[r, 2] = cu
                sched[r, 3] = b
            else:
                sched[r, 2] = -1
    return col, rho, plist, sched, nslots, maxcc


_COL_NP, _RHO_NP, _PLIST_NP, _SCHED_NP, NSLOTS, MAXCC = _static_prep()


def _idx_body(z_ref, rho_ref, col_ref, o_ref):
    dz = (Z_MAX - Z_MIN) / Z_BINS
    z = z_ref[...].astype(jnp.float32) * dz + (Z_MIN + dz / 2.0)
    theta = jnp.arctan2(z, rho_ref[...])
    sc = (H_R - 1) / (THETA_MAX - THETA_MIN)
    row = jnp.clip(jnp.round((THETA_MAX - theta) * sc), 0, H_R - 1).astype(jnp.int32)
    o_ref[...] = row * W_R + col_ref[...]


def _compute_idx(zflat, rho, colv):
    rows, cols = 16, P // 16
    return pl.pallas_call(
        _idx_body,
        grid=(B,),
        in_specs=[
            pl.BlockSpec((rows, cols), lambda b: (b, 0)),
            pl.BlockSpec((rows, cols), lambda b: (0, 0)),
            pl.BlockSpec((rows, cols), lambda b: (0, 0)),
        ],
        out_specs=pl.BlockSpec((rows, cols), lambda b: (b, 0)),
        out_shape=jax.ShapeDtypeStruct((B * rows, cols), jnp.int32),
    )(zflat.reshape(B * rows, cols), rho.reshape(rows, cols),
      colv.reshape(rows, cols)).reshape(B, P)


def _tr2_body(f_ref, o_ref):
    blk = f_ref.shape[1]
    o_ref[...] = jnp.swapaxes(f_ref[...], 0, 1).reshape(blk // 2, 2 * C)


def _sc_scatter_max(featT, idxs, plist, sched):
    mesh = plsc.VectorSubcoreMesh(core_axis_name="c", subcore_axis_name="s")
    lanes = 16

    nacc = C * H_R * UNIT_COLS // 128
    nkb = C // lanes
    qwords = H_R * UNIT_COLS * lanes

    @functools.partial(
        pl.kernel,
        mesh=mesh,
        compiler_params=pltpu.CompilerParams(needs_layout_passes=False),
        out_type=jax.ShapeDtypeStruct((B * NUM_CU * nacc * 128,), jnp.float32),
        scratch_types=[
            [pltpu.VMEM((qwords + 128,), jnp.float32) for _ in range(nkb)],
            pltpu.VMEM((MAXCC * CHUNK_PAIRS,), jnp.int32),
            pltpu.VMEM((MAXCC * CHUNK_PAIRS,), jnp.int32),
            [pltpu.VMEM((CHUNK_PX,), jnp.int32)
             for _ in range(2)],
            [pltpu.VMEM((CHUNK_PX,), jnp.int32)
             for _ in range(2)],
            [pltpu.VMEM((CHUNK_PAIRS, 2 * C), jnp.float32)
             for _ in range(2)],
            pltpu.VMEM((lanes,), jnp.int32),
            [pltpu.SemaphoreType.DMA for _ in range(4)],
        ],
    )
    def k(featT_hbm, idx_hbm, plist_hbm, sched_hbm, rv_hbm,
          accs, plbuf, pairbuf, pxbufs, idxbufs, featbufs, schedbuf, sems):
        wid = lax.axis_index("s") * 2 + lax.axis_index("c")
        ninf = jnp.full((lanes,), -jnp.inf, jnp.float32)
        dummy0 = H_R * UNIT_COLS

        chv = jnp.arange(lanes, dtype=jnp.int32)

        def slot_body(s, _):
            pltpu.sync_copy(sched_hbm.at[s * NUM_WORKERS + wid], schedbuf)
            sv = schedbuf[...]
            coff = sv[0]
            ccnt = sv[1]
            cu = sv[2]
            bb = sv[3]
            c0 = cu * UNIT_COLS

            @pl.when(cu >= 0)
            def _():
                def init_body(i, _):
                    for a in accs:
                        for u in range(128 // lanes):
                            a[pl.ds(i * 128 + u * lanes, lanes)] = ninf
                    return 0

                lax.fori_loop(0, qwords // 128 + 1, init_body, 0)

                pltpu.sync_copy(
                    plist_hbm.at[pl.ds(coff * CHUNK_PAIRS,
                                       MAXCC * CHUNK_PAIRS)], plbuf)

                def off_body(j, _):
                    pairbuf[pl.ds(j * lanes, lanes)] = (
                        plbuf[pl.ds(j * lanes, lanes)] + bb * (P // 2))
                    return 0

                lax.fori_loop(0, MAXCC * CHUNK_PAIRS // lanes, off_body, 0)

                halves = jnp.arange(lanes, dtype=jnp.int32) // 2
                parity = jnp.arange(lanes, dtype=jnp.int32) % 2

                def issue(ci, d):
                    for j in range(CHUNK_PX // lanes):
                        pv = plsc.load_gather(
                            pairbuf, [halves + (ci * CHUNK_PAIRS + j * 8)])
                        pxbufs[d][pl.ds(j * lanes, lanes)] = (
                            pv * 2 + parity)
                    pltpu.async_copy(
                        featT_hbm.at[
                            pairbuf.at[pl.ds(ci * CHUNK_PAIRS, CHUNK_PAIRS)]],
                        featbufs[d], sems[d])
                    for h in range(2):
                        pltpu.async_copy(
                            idx_hbm.at[pxbufs[d].at[pl.ds(h * 128, 128)]],
                            idxbufs[d].at[pl.ds(h * 128, 128)], sems[2 + d])

                def wait(d):
                    pltpu.make_async_copy(
                        featT_hbm.at[pairbuf.at[pl.ds(0, CHUNK_PAIRS)]],
                        featbufs[d], sems[d]).wait()
                    pltpu.make_async_copy(idx_hbm.at[pl.ds(0, CHUNK_PX)],
                                          idxbufs[d], sems[2 + d]).wait()

                issue(0, 0)

                def compute(ci, featbuf, idxbuf):
                    def group_body(j, _):
                        iv = idxbuf[pl.ds(j * lanes, lanes)]
                        rowv = lax.shift_right_logical(iv, 11)
                        collv = (iv & (W_R - 1)) - c0
                        validm = (collv >= 0) & (collv < UNIT_COLS)
                        basev = (jnp.where(validm,
                                           rowv * UNIT_COLS + collv,
                                           dummy0) * lanes)
                        for l in range(lanes):
                            o = chv + basev[l]
                            pair = j * (lanes // 2) + l // 2
                            fvs = [featbuf[pair,
                                           pl.ds((l % 2) * C + kb * lanes,
                                                 lanes)]
                                   for kb in range(nkb)]
                            curs = [plsc.load_gather(accs[kb], [o])
                                    for kb in range(nkb)]
                            for kb in range(nkb):
                                plsc.store_scatter(
                                    accs[kb], [o],
                                    jnp.maximum(curs[kb], fvs[kb]))
                        return 0

                    lax.fori_loop(0, CHUNK_PX // lanes, group_body, 0)

                def chunk_pair(ci2, _):
                    ci0 = ci2 * 2

                    @pl.when(ci0 + 1 < ccnt)
                    def _():
                        issue(ci0 + 1, 1)

                    wait(0)
                    compute(ci0, featbufs[0], idxbufs[0])

                    @pl.when(ci0 + 1 < ccnt)
                    def _():
                        @pl.when(ci0 + 2 < ccnt)
                        def _():
                            issue(ci0 + 2, 0)

                        wait(1)
                        compute(ci0 + 1, featbufs[1], idxbufs[1])
                    return 0

                lax.fori_loop(0, (ccnt + 1) // 2, chunk_pair, 0)
                ubase = (bb * NUM_CU + cu) * (nacc * 128)
                for kb in range(nkb):
                    pltpu.sync_copy(
                        accs[kb].at[pl.ds(0, qwords)],
                        rv_hbm.at[pl.ds(ubase + kb * qwords, qwords)])
            return 0

        lax.fori_loop(0, NSLOTS, slot_body, 0)

    return k(featT, idxs, plist, sched)


def _clean_body(r_ref, o_ref):
    v = r_ref[...]
    o_ref[...] = jnp.where(jnp.isneginf(v), jnp.zeros_like(v), v)


def _cleanup(rv):
    blk = 2048
    n = rv.size
    shape = rv.shape
    return pl.pallas_call(
        _clean_body,
        grid=(n // (8 * blk),),
        in_specs=[pl.BlockSpec((8, blk), lambda i: (i, 0))],
        out_specs=pl.BlockSpec((8, blk), lambda i: (i, 0)),
        out_shape=jax.ShapeDtypeStruct((n // blk, blk), jnp.float32),
    )(rv.reshape(n // blk, blk)).reshape(shape)


def _geometry():
    y_lin = jnp.linspace(YMAX, YMIN, H_B)
    x_lin = jnp.linspace(XMIN, XMAX, W_B)
    yg, xg = jnp.meshgrid(y_lin, x_lin, indexing="ij")
    phi = jnp.arctan2(yg, xg)
    colv = jnp.clip(jnp.round((phi - PHI_MIN) / (PHI_MAX - PHI_MIN) * (W_R - 1)),
                    0, W_R - 1).astype(jnp.int32).reshape(-1)
    rho = (jnp.sqrt(xg ** 2 + yg ** 2) + 1e-06).reshape(-1)
    return colv, rho


def kernel(bev_feat, bev_z_bin):
    colv, rho = _geometry()
    zflat = bev_z_bin.reshape(B, P)
    idx = _compute_idx(zflat, rho, colv)

    featT = jnp.swapaxes(bev_feat.reshape(B, C, P), 1, 2).reshape(
        B * P // 2, 2 * C)

    plist = jnp.asarray(_PLIST_NP)
    rvt = _sc_scatter_max(featT, idx.reshape(B * P), plist,
                          jnp.asarray(_SCHED_NP))
    rvt = _cleanup(rvt)
    return (rvt.reshape(B, NUM_CU, C // 16, H_R, UNIT_COLS, 16)
            .transpose(0, 2, 5, 3, 1, 4).reshape(B, C, H_R, W_R))
